# baseline (speedup 1.0000x reference)
import functools

import jax
import jax.numpy as jnp
from jax import lax
from jax.experimental import pallas as pl
from jax.experimental.pallas import tpu as pltpu
from jax.experimental.pallas import tpu_sc as plsc

NUM_RELATIONS = 1000
DIM = 32
BATCH = 16384
NC = 2
NS = 16
NW = NC * NS
B_PER_W = BATCH // NW
IDX_CHUNK = 128
N_CHUNKS = B_PER_W // IDX_CHUNK
QSTRIDE = 17


@functools.partial(
    pl.kernel,
    out_type=jax.ShapeDtypeStruct((BATCH,), jnp.float32),
    mesh=plsc.VectorSubcoreMesh(core_axis_name="c", subcore_axis_name="s"),
    compiler_params=pltpu.CompilerParams(
        needs_layout_passes=False, use_tc_tiling_on_sc=False,
        skip_device_barrier=True, disable_bounds_checks=True,
        disable_semaphore_checks=True),
    scratch_types=[
        pltpu.VMEM((N_CHUNKS, IDX_CHUNK), jnp.int32),
        pltpu.VMEM((B_PER_W, DIM), jnp.float32),
        pltpu.VMEM((B_PER_W, DIM), jnp.float32),
        pltpu.VMEM((B_PER_W * QSTRIDE,), jnp.float32),
        pltpu.VMEM((B_PER_W,), jnp.float32),
        pltpu.SemaphoreType.DMA,
        pltpu.SemaphoreType.DMA,
        pltpu.SemaphoreType.DMA,
        pltpu.SemaphoreType.DMA,
        pltpu.SemaphoreType.DMA,
    ],
)
def _dist_mult(u_hbm, r_hbm, w_hbm, out_hbm,
               idx_v, u_v, w_v, q_v, out_v, *sems):
    wid = lax.axis_index("s") * NC + lax.axis_index("c")
    base = wid * B_PER_W
    ublk = (wid // 4) * B_PER_W
    ulane = (wid % 4) * DIM

    pltpu.sync_copy(r_hbm.at[pl.ds(wid * N_CHUNKS, N_CHUNKS)], idx_v)

    def fire(s):
        off = s * IDX_CHUNK
        return [
            pltpu.async_copy(w_hbm.at[idx_v.at[s]],
                             w_v.at[pl.ds(off, IDX_CHUNK)], sems[s]),
            pltpu.async_copy(u_hbm.at[pl.ds(ublk + off, IDX_CHUNK),
                                      pl.ds(ulane, DIM)],
                             u_v.at[pl.ds(off, IDX_CHUNK)], sems[s]),
        ]

    lanes = lax.iota(jnp.int32, 16)
    qcol = lanes * QSTRIDE

    def group(g):
        rbase = g * 16
        qoff = g * (16 * QSTRIDE)
        for i in range(16):
            row = rbase + i
            u0 = u_v[row, pl.ds(0, 16)]
            u1 = u_v[row, pl.ds(16, 16)]
            w0 = w_v[row, pl.ds(0, 16)]
            w1 = w_v[row, pl.ds(16, 16)]
            q_v[pl.ds(qoff + i * QSTRIDE, 16)] = u0 * w0 + u1 * w1
        cols = [plsc.load_gather(q_v, [qoff + qcol + d]) for d in range(16)]
        while len(cols) > 1:
            cols = [cols[k] + cols[k + 1] for k in range(0, len(cols), 2)]
        out_v[pl.ds(rbase, 16)] = cols[0]

    groups_per_stage = IDX_CHUNK // 16
    out_sem = sems[N_CHUNKS]
    pending = fire(0)
    out_copies = []
    for s in range(N_CHUNKS):
        nxt = fire(s + 1) if s + 1 < N_CHUNKS else []
        for cp in pending:
            cp.wait()
        pending = nxt
        goff = s * groups_per_stage
        plsc.parallel_loop(goff, goff + groups_per_stage, unroll=2)(group)
        off = s * IDX_CHUNK
        out_copies.append(
            pltpu.async_copy(out_v.at[pl.ds(off, IDX_CHUNK)],
                             out_hbm.at[pl.ds(base + off, IDX_CHUNK)],
                             out_sem))
    for cp in out_copies:
        cp.wait()


TC_BLK = 2048


def _mul_body(h_ref, t_ref, o_ref):
    x = h_ref[...] * t_ref[...]
    quarter = TC_BLK // 4
    o_ref[...] = jnp.concatenate(
        [x[k * quarter:(k + 1) * quarter, :] for k in range(4)], axis=1)


def _mul_flat(e_h, e_t):
    grid = BATCH // TC_BLK
    return pl.pallas_call(
        _mul_body,
        grid=(grid,),
        in_specs=[pl.BlockSpec((TC_BLK, DIM), lambda i: (i, 0)),
                  pl.BlockSpec((TC_BLK, DIM), lambda i: (i, 0))],
        out_specs=pl.BlockSpec((TC_BLK * DIM // 128, 128), lambda i: (i, 0)),
        out_shape=jax.ShapeDtypeStruct((BATCH * DIM // 128, 128), jnp.float32),
    )(e_h, e_t)


def kernel(e_h, r, e_t, rel_weight):
    u = _mul_flat(e_h, e_t)
    r2 = jnp.reshape(r.astype(jnp.int32), (BATCH // IDX_CHUNK, IDX_CHUNK))
    return _dist_mult(u, r2, rel_weight)


# speedup vs baseline: 1.5984x; 1.5984x over previous
import functools

import jax
import jax.numpy as jnp
from jax import lax
from jax.experimental import pallas as pl
from jax.experimental.pallas import tpu as pltpu
from jax.experimental.pallas import tpu_sc as plsc

NUM_RELATIONS = 1000
REL_PAD = 1024
DIM = 32
BATCH = 16384
NC = 2
NS = 16
NW = NC * NS
B_PER_W = BATCH // NW
E_WORDS = B_PER_W * DIM
W_WORDS = REL_PAD * DIM
TI_STRIDE = BATCH * 8


def _native_flat(x, rows):
    a = jnp.reshape(jnp.transpose(x), (4, 8, rows // 128, 128))
    return jnp.reshape(jnp.transpose(a, (0, 2, 1, 3)), (rows * DIM,))


@functools.partial(
    pl.kernel,
    out_type=jax.ShapeDtypeStruct((BATCH,), jnp.float32),
    mesh=plsc.VectorSubcoreMesh(core_axis_name="c", subcore_axis_name="s"),
    compiler_params=pltpu.CompilerParams(
        needs_layout_passes=False, use_tc_tiling_on_sc=False,
        skip_device_barrier=True, disable_bounds_checks=True,
        disable_semaphore_checks=True),
    scratch_types=[
        pltpu.VMEM((B_PER_W,), jnp.int32),
        pltpu.VMEM((E_WORDS,), jnp.float32),
        pltpu.VMEM((W_WORDS,), jnp.float32),
        pltpu.VMEM((E_WORDS,), jnp.float32),
        pltpu.VMEM((B_PER_W,), jnp.float32),
        pltpu.SemaphoreType.DMA,
    ],
)
def _dist_mult(h_hbm, r_hbm, t_hbm, w_hbm, out_hbm,
               idx_v, h_v, w_v, t_v, out_v, sem):
    wid = lax.axis_index("s") * NC + lax.axis_index("c")
    base = wid * B_PER_W

    copies = [pltpu.async_copy(w_hbm, w_v, sem)]
    for ti in range(4):
        src = ti * TI_STRIDE + wid * 4096
        copies.append(pltpu.async_copy(
            h_hbm.at[pl.ds(src, 4096)], h_v.at[pl.ds(ti * 4096, 4096)], sem))
        copies.append(pltpu.async_copy(
            t_hbm.at[pl.ds(src, 4096)], t_v.at[pl.ds(ti * 4096, 4096)], sem))
    pltpu.sync_copy(r_hbm.at[pl.ds(base, B_PER_W)], idx_v)
    for cp in copies:
        cp.wait()

    def group(g):
        r16 = idx_v[pl.ds(g * 16, 16)]
        wrow = ((r16 >> 7) << 10) + (r16 & 127)
        ebase = (g // 8) * 1024 + (g % 8) * 16
        accs = [jnp.zeros((16,), jnp.float32) for _ in range(4)]
        for d in range(DIM):
            eoff = (d // 8) * 4096 + (d % 8) * 128 + ebase
            woff = (d // 8) * 8192 + (d % 8) * 128
            h = h_v[pl.ds(eoff, 16)]
            t = t_v[pl.ds(eoff, 16)]
            w = plsc.load_gather(w_v, [wrow + woff])
            accs[d % 4] = accs[d % 4] + h * w * t
        out_v[pl.ds(g * 16, 16)] = (accs[0] + accs[1]) + (accs[2] + accs[3])

    plsc.parallel_loop(0, B_PER_W // 16, unroll=2)(group)
    pltpu.sync_copy(out_v, out_hbm.at[pl.ds(base, B_PER_W)])


def kernel(e_h, r, e_t, rel_weight):
    w_pad = jnp.zeros((REL_PAD, DIM), jnp.float32).at[:NUM_RELATIONS].set(
        rel_weight)
    return _dist_mult(
        _native_flat(e_h, BATCH),
        r.astype(jnp.int32),
        _native_flat(e_t, BATCH),
        _native_flat(w_pad, REL_PAD),
    )


# speedup vs baseline: 1.6359x; 1.0235x over previous
import functools

import jax
import jax.numpy as jnp
from jax import lax
from jax.experimental import pallas as pl
from jax.experimental.pallas import tpu as pltpu
from jax.experimental.pallas import tpu_sc as plsc

NUM_RELATIONS = 1000
REL_PAD = 1024
DIM = 32
BATCH = 16384
NC = 2
NS = 16
NW = NC * NS
B_PER_W = BATCH // NW
E_WORDS = B_PER_W * DIM
W_WORDS = REL_PAD * DIM
TI_STRIDE = BATCH * 8


def _native_flat(x, rows):
    a = jnp.reshape(jnp.transpose(x), (4, 8, rows // 128, 128))
    return jnp.reshape(jnp.transpose(a, (0, 2, 1, 3)), (rows * DIM,))


@functools.partial(
    pl.kernel,
    out_type=jax.ShapeDtypeStruct((BATCH,), jnp.float32),
    mesh=plsc.VectorSubcoreMesh(core_axis_name="c", subcore_axis_name="s"),
    compiler_params=pltpu.CompilerParams(
        needs_layout_passes=False, use_tc_tiling_on_sc=False,
        skip_device_barrier=True, disable_bounds_checks=True,
        disable_semaphore_checks=True),
    scratch_types=[
        pltpu.VMEM((B_PER_W,), jnp.int32),
        pltpu.VMEM((E_WORDS,), jnp.float32),
        pltpu.VMEM((W_WORDS,), jnp.float32),
        pltpu.VMEM((E_WORDS,), jnp.float32),
        pltpu.VMEM((B_PER_W,), jnp.float32),
        pltpu.SemaphoreType.DMA,
        pltpu.SemaphoreType.DMA,
        pltpu.SemaphoreType.DMA,
        pltpu.SemaphoreType.DMA,
    ],
)
def _dist_mult(h_hbm, r_hbm, t_hbm, w_hbm, out_hbm,
               idx_v, h_v, w_v, t_v, out_v, *sems):
    wid = lax.axis_index("s") * NC + lax.axis_index("c")
    base = wid * B_PER_W

    copies = []
    for ti in range(4):
        src = ti * TI_STRIDE + wid * 4096
        copies.append([
            pltpu.async_copy(w_hbm.at[pl.ds(ti * 8192, 8192)],
                             w_v.at[pl.ds(ti * 8192, 8192)], sems[ti]),
            pltpu.async_copy(h_hbm.at[pl.ds(src, 4096)],
                             h_v.at[pl.ds(ti * 4096, 4096)], sems[ti]),
            pltpu.async_copy(t_hbm.at[pl.ds(src, 4096)],
                             t_v.at[pl.ds(ti * 4096, 4096)], sems[ti]),
        ])
    pltpu.sync_copy(r_hbm.at[pl.ds(base, B_PER_W)], idx_v)

    def make_pass(ti):
        def group(g):
            r16 = idx_v[pl.ds(g * 16, 16)]
            wrow = ((r16 >> 7) << 10) + (r16 & 127)
            ebase = (g // 8) * 1024 + (g % 8) * 16
            accs = [jnp.zeros((16,), jnp.float32) for _ in range(2)]
            for dd in range(8):
                eoff = ti * 4096 + dd * 128 + ebase
                woff = ti * 8192 + dd * 128
                h = h_v[pl.ds(eoff, 16)]
                t = t_v[pl.ds(eoff, 16)]
                w = plsc.load_gather(w_v, [wrow + woff])
                accs[dd % 2] = accs[dd % 2] + h * w * t
            acc = accs[0] + accs[1]
            ds = pl.ds(g * 16, 16)
            if ti == 0:
                out_v[ds] = acc
            else:
                out_v[ds] = out_v[ds] + acc
        return group

    for ti in range(4):
        for cp in copies[ti]:
            cp.wait()
        plsc.parallel_loop(0, B_PER_W // 16, unroll=2)(make_pass(ti))
    pltpu.sync_copy(out_v, out_hbm.at[pl.ds(base, B_PER_W)])


def kernel(e_h, r, e_t, rel_weight):
    w_pad = jnp.zeros((REL_PAD, DIM), jnp.float32).at[:NUM_RELATIONS].set(
        rel_weight)
    return _dist_mult(
        _native_flat(e_h, BATCH),
        r.astype(jnp.int32),
        _native_flat(e_t, BATCH),
        _native_flat(w_pad, REL_PAD),
    )
